# Initial kernel scaffold; baseline (speedup 1.0000x reference)
import jax
import jax.numpy as jnp
from jax.experimental import pallas as pl


def kernel(feat_prop, neigh_idx, W, b):
    raise NotImplementedError("write your pallas kernel here")


# speedup vs baseline: 2.8410x; 2.8410x over previous
import functools

import jax
import jax.numpy as jnp
from jax import lax
from jax.experimental import pallas as pl
from jax.experimental.pallas import tpu as pltpu
from jax.experimental.pallas import tpu_sc as plsc

_N, _K, _D, _OUT = 10000, 16, 256, 256
_L = 16
_NC, _NS = 2, 16
_NW = _NC * _NS
_CHUNK = 320
_NB = 8
_NCH = _D // _L
_DE = _D + _L


def _inv_body(feat_ref, inv_ref):
    x = feat_ref[...]
    inv_ref[...] = lax.rsqrt(jnp.sum(x * x, axis=1, keepdims=True))


_inv_call = pl.pallas_call(
    _inv_body,
    out_shape=jax.ShapeDtypeStruct((_N, 1), jnp.float32),
)


_mesh = plsc.VectorSubcoreMesh(core_axis_name="c", subcore_axis_name="s")


@functools.partial(
    pl.kernel,
    mesh=_mesh,
    compiler_params=pltpu.CompilerParams(needs_layout_passes=False),
    out_type=jax.ShapeDtypeStruct((_N, _DE), jnp.float32),
    scratch_types=[
        pltpu.VMEM((_N,), jnp.float32),
        pltpu.VMEM((_NB * _K,), jnp.int32),
        pltpu.VMEM((_NB * _K, _D), jnp.float32),
        pltpu.VMEM((_NB, _D), jnp.float32),
        pltpu.VMEM((_NB, _DE), jnp.float32),
        pltpu.SemaphoreType.DMA,
    ],
)
def _neigh_sc(feat_hbm, idxf_hbm, inv_hbm, agg_hbm,
              inv_v, idx_v, rows_v, cen_v, acc_v, sem):
    wid = lax.axis_index("s") * _NC + lax.axis_index("c")
    base = wid * _CHUNK
    rows_w = jnp.minimum(_CHUNK, _N - base)
    nblk = rows_w // _NB

    pltpu.sync_copy(inv_hbm, inv_v)
    e0 = (lax.iota(jnp.int32, _L) == 0).astype(jnp.float32)

    def blk_body(blk, carry):
        rowbase = base + blk * _NB
        pltpu.sync_copy(idxf_hbm.at[pl.ds(rowbase * _K, _NB * _K)], idx_v)
        pltpu.sync_copy(feat_hbm.at[pl.ds(rowbase, _NB)], cen_v)
        pltpu.async_copy(feat_hbm.at[idx_v], rows_v, sem).wait()

        def node_body(j, c2):
            r0 = j * _K
            kidx = idx_v[pl.ds(r0, _K)]
            invk = plsc.load_gather(inv_v, [kidx])
            ctr_idx = jnp.full((_L,), rowbase + j, dtype=jnp.int32)
            inv_n = plsc.load_gather(inv_v, [ctr_idx])
            wscale = invk * inv_n
            cen = [cen_v[j, pl.ds(i * _L, _L)] for i in range(_NCH)]
            acc = [jnp.zeros((_L,), jnp.float32) for _ in range(_NCH)]
            acc_s = jnp.zeros((_L,), jnp.float32)
            for k in range(_K):
                row = [rows_v[r0 + k, pl.ds(i * _L, _L)] for i in range(_NCH)]
                p = row[0] * cen[0]
                for i in range(1, _NCH):
                    p = p + row[i] * cen[i]
                w = jnp.sum(p) * wscale[k]
                for i in range(_NCH):
                    acc[i] = acc[i] + w * row[i]
                acc_s = acc_s + w * e0
            for i in range(_NCH):
                acc_v[j, pl.ds(i * _L, _L)] = acc[i]
            acc_v[j, pl.ds(_D, _L)] = acc_s
            return c2

        lax.fori_loop(0, _NB, node_body, 0)
        pltpu.sync_copy(acc_v, agg_hbm.at[pl.ds(rowbase, _NB)])
        return carry

    lax.fori_loop(0, nblk, blk_body, 0)


def _fin_body(agg_ref, feat_ref, w_ref, b_ref, out_ref):
    w1 = w_ref[:, :_D]
    w2 = w_ref[:, _D:]
    agg = agg_ref[:, :_D]
    sw = agg_ref[:, _D:_D + 1]
    dn = (((1,), (1,)), ((), ()))
    p = lax.dot_general(feat_ref[...], w2, dn,
                        preferred_element_type=jnp.float32) + b_ref[...]
    a = lax.dot_general(agg, w1, dn,
                        preferred_element_type=jnp.float32)
    out_ref[...] = (a + sw * p) * (1.0 / _K)


_fin_call = pl.pallas_call(
    _fin_body,
    out_shape=jax.ShapeDtypeStruct((_N, _OUT), jnp.float32),
)


def kernel(feat_prop, neigh_idx, W, b):
    idx_flat = neigh_idx.astype(jnp.int32).reshape(-1)
    inv = _inv_call(feat_prop)
    agg_ext = _neigh_sc(feat_prop, idx_flat, inv.reshape(_N))
    return _fin_call(agg_ext, feat_prop, W, b.reshape(1, _OUT))
